# baseline (speedup 1.0000x reference)
import math

import jax
import jax.numpy as jnp
from jax import lax
from jax.experimental import pallas as pl
from jax.experimental.pallas import tpu as pltpu
from jax.experimental.pallas import tpu_sc as plsc

_NC = 2
_NS = 16
_NW = _NC * _NS
_L = 16

_C = 8192
_MAGIC = float(2 ** 23)


def _body_fn(npts, nchunk, grid_shape, nwords):
    pts_per_worker = npts // _NW
    ncells = grid_shape[0] * grid_shape[1] * grid_shape[2]
    sj = grid_shape[2]
    si = grid_shape[1] * grid_shape[2]
    stage_w = nwords // _NS

    def body(xyz_hbm, maskw_hbm, params_hbm, out_hbm,
             params_v, xyz_v, idx_v, enc_v, words_v, outw_v, spmem, sem):
        sid = lax.axis_index("s")
        wid = sid * _NC + lax.axis_index("c")

        @pl.loop(0, stage_w // _C)
        def _stage(t):
            off = sid * stage_w + t * _C
            pltpu.sync_copy(maskw_hbm.at[pl.ds(off, _C)], words_v)
            pltpu.sync_copy(words_v, spmem.at[pl.ds(off, _C)])
        plsc.subcore_barrier()

        pltpu.sync_copy(params_hbm, params_v)
        sx = params_v[pl.ds(0 * _L, _L)]
        sy = params_v[pl.ds(1 * _L, _L)]
        sz = params_v[pl.ds(2 * _L, _L)]
        tx = params_v[pl.ds(3 * _L, _L)]
        ty = params_v[pl.ds(4 * _L, _L)]
        tz = params_v[pl.ds(5 * _L, _L)]
        lane12 = lax.iota(jnp.int32, _L) * 12

        @pl.loop(0, nchunk)
        def _chunk(n):
            pt0 = wid * pts_per_worker + n * _C
            out0 = wid * (pts_per_worker // 4) + n * (_C // 4)
            pltpu.sync_copy(xyz_hbm.at[pl.ds(pt0 * 3, 3 * _C)], xyz_v)

            @pl.loop(0, _C // 64)
            def _pass1(b):
                for c in range(4):
                    ix3 = lane12 + (b * 192 + 3 * c)
                    x = plsc.load_gather(xyz_v, [ix3])
                    y = plsc.load_gather(xyz_v, [ix3 + 1])
                    z = plsc.load_gather(xyz_v, [ix3 + 2])
                    ri = (x * sx + tx + _MAGIC) - _MAGIC
                    rj = (y * sy + ty + _MAGIC) - _MAGIC
                    rk = (z * sz + tz + _MAGIC) - _MAGIC
                    ii = ri.astype(jnp.int32)
                    jj = rj.astype(jnp.int32)
                    kk = rk.astype(jnp.int32)
                    ok = ((ii >= 0) & (ii < grid_shape[0])
                          & (jj >= 0) & (jj < grid_shape[1])
                          & (kk >= 0) & (kk < grid_shape[2]))
                    lin = ii * si + jj * sj + kk
                    lin = jnp.clip(lin, 0, ncells - 1)
                    enc = (lin & 31) | (ok.astype(jnp.int32) << 5)
                    pos = b * 64 + c * 16
                    idx_v[pos // 128, pl.ds(pos % 128, _L)] = lin >> 5
                    enc_v[pl.ds(pos, _L)] = enc

            @pl.loop(0, _C // 128, step=8)
            def _gather(j0):
                for t in range(8):
                    j = j0 + t
                    pltpu.async_copy(spmem.at[idx_v.at[j]],
                                     words_v.at[pl.ds(j * 128, 128)], sem)

            pltpu.make_async_copy(maskw_hbm.at[pl.ds(0, _C)],
                                  words_v, sem).wait()

            @pl.loop(0, _C // 64)
            def _pass2(b):
                acc = None
                for c in range(4):
                    pos = b * 64 + c * 16
                    w = words_v[pl.ds(pos, _L)]
                    e = enc_v[pl.ds(pos, _L)]
                    bit = (w >> (e & 31)) & (e >> 5) & 1
                    term = bit << (8 * c) if c else bit
                    acc = term if acc is None else acc | term
                outw_v[pl.ds(b * 16, _L)] = acc

            pltpu.sync_copy(outw_v, out_hbm.at[pl.ds(out0, _C // 4)])

    return body


def kernel(xyz, mask, xyz_min, xyz_max):
    out_shape = xyz.shape[:-1]
    npts = math.prod(out_shape)
    xyz_flat = xyz.reshape(-1)

    w4 = lax.bitcast_convert_type(
        mask.astype(jnp.uint8).reshape(-1, 4), jnp.int32)
    nib = ((w4 * 0x01020408) >> 24) & 0xF
    shifts = jnp.arange(8, dtype=jnp.int32) * 4
    maskw = (nib.reshape(-1, 8) << shifts).sum(axis=1, dtype=jnp.int32)
    nwords = maskw.shape[0]

    grid_f = jnp.asarray(mask.shape, jnp.float32)
    scale = (grid_f - 1.0) / (xyz_max.astype(jnp.float32)
                              - xyz_min.astype(jnp.float32))
    shift = -xyz_min.astype(jnp.float32) * scale
    params = jnp.repeat(jnp.concatenate([scale, shift]), _L)
    nchunk = npts // (_NW * _C)

    outw = pl.kernel(
        _body_fn(npts, nchunk, mask.shape, nwords),
        out_type=jax.ShapeDtypeStruct((npts // 4,), jnp.int32),
        mesh=plsc.VectorSubcoreMesh(
            core_axis_name="c", subcore_axis_name="s",
            num_cores=_NC, num_subcores=_NS),
        compiler_params=pltpu.CompilerParams(needs_layout_passes=False),
        scratch_types=[
            pltpu.VMEM((6 * _L,), jnp.float32),
            pltpu.VMEM((3 * _C,), jnp.float32),
            pltpu.VMEM((_C // 128, 128), jnp.int32),
            pltpu.VMEM((_C,), jnp.int32),
            pltpu.VMEM((_C,), jnp.int32),
            pltpu.VMEM((_C // 4,), jnp.int32),
            pltpu.VMEM_SHARED((nwords,), jnp.int32),
            pltpu.SemaphoreType.DMA,
        ],
    )(xyz_flat, maskw, params)

    out_bytes = lax.bitcast_convert_type(outw, jnp.uint8)
    return out_bytes.reshape(out_shape) != 0


# speedup vs baseline: 10.1991x; 10.1991x over previous
import math

import jax
import jax.numpy as jnp
from jax import lax
from jax.experimental import pallas as pl
from jax.experimental.pallas import tpu as pltpu
from jax.experimental.pallas import tpu_sc as plsc

_NC = 2
_NS = 16
_NW = _NC * _NS
_L = 16

_C = 8192
_MAGIC = float(2 ** 23)


def _body_fn(npts, nchunk, grid_shape, nwords):
    pts_per_worker = npts // _NW
    ncells = grid_shape[0] * grid_shape[1] * grid_shape[2]
    sj = grid_shape[2]
    si = grid_shape[1] * grid_shape[2]
    stage_w = nwords // _NS

    def body(x_hbm, y_hbm, z_hbm, maskw_hbm, params_hbm, out_hbm,
             params_v, xs_v, ys_v, zs_v, idx_v, enc_v, words_v, outw_v,
             spmem, sem):
        sid = lax.axis_index("s")
        wid = sid * _NC + lax.axis_index("c")

        @pl.loop(0, stage_w // _C)
        def _stage(t):
            off = sid * stage_w + t * _C
            pltpu.sync_copy(maskw_hbm.at[pl.ds(off, _C)], words_v)
            pltpu.sync_copy(words_v, spmem.at[pl.ds(off, _C)])
        plsc.subcore_barrier()

        pltpu.sync_copy(params_hbm, params_v)
        sx = params_v[pl.ds(0 * _L, _L)]
        sy = params_v[pl.ds(1 * _L, _L)]
        sz = params_v[pl.ds(2 * _L, _L)]
        tx = params_v[pl.ds(3 * _L, _L)]
        ty = params_v[pl.ds(4 * _L, _L)]
        tz = params_v[pl.ds(5 * _L, _L)]

        @pl.loop(0, nchunk)
        def _chunk(n):
            pt0 = wid * pts_per_worker + n * _C
            pltpu.async_copy(x_hbm.at[pl.ds(pt0, _C)], xs_v, sem)
            pltpu.async_copy(y_hbm.at[pl.ds(pt0, _C)], ys_v, sem)
            pltpu.async_copy(z_hbm.at[pl.ds(pt0, _C)], zs_v, sem)
            pltpu.make_async_copy(x_hbm.at[pl.ds(pt0, _C)], xs_v, sem).wait()
            pltpu.make_async_copy(y_hbm.at[pl.ds(pt0, _C)], ys_v, sem).wait()
            pltpu.make_async_copy(z_hbm.at[pl.ds(pt0, _C)], zs_v, sem).wait()

            @pl.loop(0, _C // 128)
            def _pass1(b):
                for c in range(8):
                    pos = b * 128 + c * _L
                    x = xs_v[pl.ds(pos, _L)]
                    y = ys_v[pl.ds(pos, _L)]
                    z = zs_v[pl.ds(pos, _L)]
                    ri = (x * sx + tx + _MAGIC) - _MAGIC
                    rj = (y * sy + ty + _MAGIC) - _MAGIC
                    rk = (z * sz + tz + _MAGIC) - _MAGIC
                    ii = ri.astype(jnp.int32)
                    jj = rj.astype(jnp.int32)
                    kk = rk.astype(jnp.int32)
                    ok = ((ii >= 0) & (ii < grid_shape[0])
                          & (jj >= 0) & (jj < grid_shape[1])
                          & (kk >= 0) & (kk < grid_shape[2]))
                    lin = ii * si + jj * sj + kk
                    lin = jnp.clip(lin, 0, ncells - 1)
                    enc = (lin & 31) | (ok.astype(jnp.int32) << 5)
                    idx_v[b, pl.ds(c * _L, _L)] = lin >> 5
                    enc_v[pl.ds(pos, _L)] = enc

            @pl.loop(0, _C // 128, step=8)
            def _gather(j0):
                for t in range(8):
                    j = j0 + t
                    pltpu.async_copy(spmem.at[idx_v.at[j]],
                                     words_v.at[pl.ds(j * 128, 128)], sem)

            pltpu.make_async_copy(maskw_hbm.at[pl.ds(0, _C)],
                                  words_v, sem).wait()

            @pl.loop(0, _C // 128)
            def _pass2(b):
                for c in range(8):
                    pos = b * 128 + c * _L
                    w = words_v[pl.ds(pos, _L)]
                    e = enc_v[pl.ds(pos, _L)]
                    outw_v[pl.ds(pos, _L)] = (w >> (e & 31)) & (e >> 5) & 1

            pltpu.sync_copy(outw_v, out_hbm.at[pl.ds(pt0, _C)])

    return body


def kernel(xyz, mask, xyz_min, xyz_max):
    out_shape = xyz.shape[:-1]
    npts = math.prod(out_shape)
    x = xyz[..., 0].reshape(-1)
    y = xyz[..., 1].reshape(-1)
    z = xyz[..., 2].reshape(-1)

    m = mask.reshape(-1, 32).astype(jnp.int32)
    maskw = (m << jnp.arange(32, dtype=jnp.int32)).sum(
        axis=1, dtype=jnp.int32)
    nwords = maskw.shape[0]

    grid_f = jnp.asarray(mask.shape, jnp.float32)
    scale = (grid_f - 1.0) / (xyz_max.astype(jnp.float32)
                              - xyz_min.astype(jnp.float32))
    shift = -xyz_min.astype(jnp.float32) * scale
    params = jnp.repeat(jnp.concatenate([scale, shift]), _L)
    nchunk = npts // (_NW * _C)

    outw = pl.kernel(
        _body_fn(npts, nchunk, mask.shape, nwords),
        out_type=jax.ShapeDtypeStruct((npts,), jnp.int32),
        mesh=plsc.VectorSubcoreMesh(
            core_axis_name="c", subcore_axis_name="s",
            num_cores=_NC, num_subcores=_NS),
        compiler_params=pltpu.CompilerParams(needs_layout_passes=False),
        scratch_types=[
            pltpu.VMEM((6 * _L,), jnp.float32),
            pltpu.VMEM((_C,), jnp.float32),
            pltpu.VMEM((_C,), jnp.float32),
            pltpu.VMEM((_C,), jnp.float32),
            pltpu.VMEM((_C // 128, 128), jnp.int32),
            pltpu.VMEM((_C,), jnp.int32),
            pltpu.VMEM((_C,), jnp.int32),
            pltpu.VMEM((_C,), jnp.int32),
            pltpu.VMEM_SHARED((nwords,), jnp.int32),
            pltpu.SemaphoreType.DMA,
        ],
    )(x, y, z, maskw, params)

    return outw.reshape(out_shape) != 0
